# baseline (speedup 1.0000x reference)
import functools

import jax
import jax.numpy as jnp
from jax import lax
from jax.experimental import pallas as pl
from jax.experimental.pallas import tpu as pltpu
from jax.experimental.pallas import tpu_sc as plsc

_B = 16
_H = 1024
_TOTAL = 16384
_TWOH = 2 * _H
_TILE_M = 512
_M_TILES = _TOTAL // _TILE_M
_PREP_TILES = 4
_PREP_N = _TWOH // _PREP_TILES
_STEPS = _PREP_TILES + _M_TILES + 1

_NC = 2
_NS = 16
_NW = _NC * _NS
_LANES = 16
_CHUNK = _TOTAL // _NW
_NV = _CHUNK // _LANES


def _fused_kernel(culow_v_ref, cuhigh_v_ref, culow_c_ref, cuhigh_c_ref,
                  w1_ref, w2_ref, cur_ref, b1_ref,
                  b2_ref, succ_ref, w3_ref, vw1_ref, vb1_ref, vw2_ref,
                  vb2_ref, logits_ref, value_ref, rtab_ref,
                  w12b_scr, a_scr, psum_scr):
    s = pl.program_id(0)

    @pl.when(s == 0)
    def _value():
        v = jnp.dot(cur_ref[...], vw1_ref[...], preferred_element_type=jnp.float32)
        v = jnp.maximum(v + vb1_ref[...], 0.0)
        value_ref[...] = jnp.sum(v * vw2_ref[...], axis=1, keepdims=True) + vb2_ref[0, 0]

    @pl.when(s < _PREP_TILES)
    def _prep():
        w2j = w2_ref[...]
        w12b = jnp.dot(w1_ref[_H:, :], w2j, preferred_element_type=jnp.float32)
        w12b_scr[:, pl.ds(s * _PREP_N, _PREP_N)] = w12b.astype(jnp.bfloat16)
        top = jnp.dot(cur_ref[...], w1_ref[:_H, :], preferred_element_type=jnp.float32)
        top = top + b1_ref[...]
        a_scr[:, pl.ds(s * _PREP_N, _PREP_N)] = (
            jnp.dot(top, w2j, preferred_element_type=jnp.float32) + b2_ref[...])

    @pl.when((s >= _PREP_TILES) & (s < _PREP_TILES + _M_TILES))
    def _main():
        i = s - _PREP_TILES
        succ16 = succ_ref[...].astype(jnp.bfloat16)
        h = jnp.dot(succ16, w12b_scr[...], preferred_element_type=jnp.float32)
        pos = i * _TILE_M + jax.lax.broadcasted_iota(jnp.int32, (_TILE_M, 1), 0)
        onehot = ((pos >= culow_v_ref[...]) & (pos < cuhigh_v_ref[...])).astype(jnp.float32)
        h = h + jnp.dot(onehot, a_scr[...], preferred_element_type=jnp.float32)
        h = jnp.maximum(h, 0.0)
        l_col = jnp.sum(h * w3_ref[...], axis=1, keepdims=True)
        logits_ref[...] = l_col
        posT = i * _TILE_M + jax.lax.broadcasted_iota(jnp.int32, (1, _TILE_M), 1)
        onehotT = ((posT >= culow_c_ref[...]) & (posT < cuhigh_c_ref[...])).astype(jnp.float32)
        part = jnp.dot(onehotT, jnp.exp(l_col), preferred_element_type=jnp.float32)

        @pl.when(s == _PREP_TILES)
        def _init():
            psum_scr[...] = part

        @pl.when(s > _PREP_TILES)
        def _acc():
            psum_scr[...] = psum_scr[...] + part

    @pl.when(s == _STEPS - 1)
    def _recip():
        rtab_ref[...] = (1.0 / psum_scr[...]) * jnp.ones((_B, _B), jnp.float32)


def _sc_normalize_body(logits_hbm, pos_hbm, lob_hbm, hib_hbm, rtab_hbm,
                       out_hbm, x_v, pos_v, lob_v, hib_v, rt_v, sem):
    wid = lax.axis_index("s") * _NC + lax.axis_index("c")
    base = wid * _CHUNK
    d1 = pltpu.async_copy(logits_hbm.at[pl.ds(base, _CHUNK)], x_v, sem)
    d2 = pltpu.async_copy(pos_hbm.at[pl.ds(base, _CHUNK)], pos_v, sem)
    d3 = pltpu.async_copy(lob_hbm, lob_v, sem)
    d4 = pltpu.async_copy(hib_hbm, hib_v, sem)
    d5 = pltpu.async_copy(rtab_hbm, rt_v, sem)
    d1.wait()
    d2.wait()
    d3.wait()
    d4.wait()
    d5.wait()
    lob = [lob_v[pl.ds(b * _LANES, _LANES)] for b in range(_B)]
    hib = [hib_v[pl.ds(b * _LANES, _LANES)] for b in range(_B)]
    rrow = [rt_v[pl.ds(b * _LANES, _LANES)] for b in range(_B)]
    zero = jnp.full((_LANES,), 0.0, jnp.float32)
    for v in range(_NV):
        sl = pl.ds(v * _LANES, _LANES)
        pos = pos_v[sl]
        rv = zero
        for b in range(_B):
            m = (pos >= lob[b]) & (pos < hib[b])
            rv = jnp.where(m, rrow[b], rv)
        x_v[sl] = jnp.exp(x_v[sl]) * rv
    pltpu.sync_copy(x_v, out_hbm.at[pl.ds(base, _CHUNK)])


_sc_normalize = functools.partial(
    pl.kernel,
    out_type=jax.ShapeDtypeStruct((_TOTAL,), jnp.float32),
    mesh=plsc.VectorSubcoreMesh(core_axis_name="c", subcore_axis_name="s"),
    scratch_types=[
        pltpu.VMEM((_CHUNK,), jnp.float32),
        pltpu.VMEM((_CHUNK,), jnp.int32),
        pltpu.VMEM((_B * _LANES,), jnp.int32),
        pltpu.VMEM((_B * _LANES,), jnp.int32),
        pltpu.VMEM((_B * _LANES,), jnp.float32),
        pltpu.SemaphoreType.DMA,
    ],
)(_sc_normalize_body)


def kernel(current_embedding, successor_flat, cu_seqlens,
           W1, b1, W2, b2, W3, b3, Vw1, Vb1, Vw2, Vb2):
    del b3
    cu = cu_seqlens.astype(jnp.int32)
    culow_v = cu[:-1].reshape(1, _B)
    cuhigh_v = cu[1:].reshape(1, _B)
    culow_c = cu[:-1].reshape(_B, 1)
    cuhigh_c = cu[1:].reshape(_B, 1)
    b1r = b1.reshape(1, _TWOH)
    b2r = b2.reshape(1, _TWOH)
    w3r = W3.reshape(1, _TWOH)
    vb1r = Vb1.reshape(1, _H)
    vw2r = Vw2.reshape(1, _H)
    vb2r = Vb2.reshape(1, 1)

    logits_col, state_value, rtab = pl.pallas_call(
        _fused_kernel,
        grid=(_STEPS,),
        in_specs=[
            pl.BlockSpec((1, _B), lambda s: (0, 0)),
            pl.BlockSpec((1, _B), lambda s: (0, 0)),
            pl.BlockSpec((_B, 1), lambda s: (0, 0)),
            pl.BlockSpec((_B, 1), lambda s: (0, 0)),
            pl.BlockSpec((_TWOH, _TWOH), lambda s: (0, 0)),
            pl.BlockSpec((_TWOH, _PREP_N),
                         lambda s: (0, jnp.minimum(s, _PREP_TILES - 1))),
            pl.BlockSpec((_B, _H), lambda s: (0, 0)),
            pl.BlockSpec((1, _TWOH), lambda s: (0, 0)),
            pl.BlockSpec((1, _PREP_N),
                         lambda s: (0, jnp.minimum(s, _PREP_TILES - 1))),
            pl.BlockSpec((_TILE_M, _H),
                         lambda s: (jnp.clip(s - _PREP_TILES, 0, _M_TILES - 1), 0)),
            pl.BlockSpec((1, _TWOH), lambda s: (0, 0)),
            pl.BlockSpec((_H, _H), lambda s: (0, 0)),
            pl.BlockSpec((1, _H), lambda s: (0, 0)),
            pl.BlockSpec((1, _H), lambda s: (0, 0)),
            pl.BlockSpec(memory_space=pltpu.SMEM),
        ],
        out_specs=[
            pl.BlockSpec((_TILE_M, 1),
                         lambda s: (jnp.clip(s - _PREP_TILES, 0, _M_TILES - 1), 0)),
            pl.BlockSpec((_B, 1), lambda s: (0, 0)),
            pl.BlockSpec((_B, _B), lambda s: (0, 0)),
        ],
        out_shape=[
            jax.ShapeDtypeStruct((_TOTAL, 1), jnp.float32),
            jax.ShapeDtypeStruct((_B, 1), jnp.float32),
            jax.ShapeDtypeStruct((_B, _B), jnp.float32),
        ],
        scratch_shapes=[
            pltpu.VMEM((_H, _TWOH), jnp.bfloat16),
            pltpu.VMEM((_B, _TWOH), jnp.float32),
            pltpu.VMEM((_B, 1), jnp.float32),
        ],
    )(culow_v, cuhigh_v, culow_c, cuhigh_c, W1, W2, current_embedding,
      b1r, b2r, successor_flat, w3r, Vw1, vb1r, vw2r, vb2r)

    pos_tab = jnp.arange(_TOTAL, dtype=jnp.int32)
    lob_tab = jnp.repeat(cu[:-1], _LANES)
    hib_tab = jnp.repeat(cu[1:], _LANES)
    probs = _sc_normalize(logits_col.reshape(_TOTAL), pos_tab,
                          lob_tab, hib_tab, rtab.reshape(_B * _LANES))

    return probs, state_value


# speedup vs baseline: 1.0474x; 1.0474x over previous
import functools

import jax
import jax.numpy as jnp
from jax import lax
from jax.experimental import pallas as pl
from jax.experimental.pallas import tpu as pltpu
from jax.experimental.pallas import tpu_sc as plsc

_B = 16
_H = 1024
_TOTAL = 16384
_TWOH = 2 * _H
_TILE_M = 1024
_M_TILES = _TOTAL // _TILE_M
_PREP_TILES = 4
_PREP_N = _TWOH // _PREP_TILES
_STEPS = _PREP_TILES + _M_TILES + 1

_NC = 2
_NS = 16
_NW = _NC * _NS
_LANES = 16
_CHUNK = _TOTAL // _NW
_NV = _CHUNK // _LANES


def _fused_kernel(culow_v_ref, cuhigh_v_ref, culow_c_ref, cuhigh_c_ref,
                  w1_ref, w2_ref, cur_ref, b1_ref,
                  b2_ref, succ_ref, w3_ref, vw1_ref, vb1_ref, vw2_ref,
                  vb2_ref, logits_ref, value_ref, rtab_ref,
                  w12b_scr, a_scr, psum_scr):
    s = pl.program_id(0)

    @pl.when(s == 0)
    def _value():
        v = jnp.dot(cur_ref[...], vw1_ref[...], preferred_element_type=jnp.float32)
        v = jnp.maximum(v + vb1_ref[...], 0.0)
        value_ref[...] = jnp.sum(v * vw2_ref[...], axis=1, keepdims=True) + vb2_ref[0, 0]

    @pl.when(s < _PREP_TILES)
    def _prep():
        w2j = w2_ref[...]
        w12b = jnp.dot(w1_ref[_H:, :], w2j, preferred_element_type=jnp.float32)
        w12b_scr[:, pl.ds(s * _PREP_N, _PREP_N)] = w12b.astype(jnp.bfloat16)
        top = jnp.dot(cur_ref[...], w1_ref[:_H, :], preferred_element_type=jnp.float32)
        top = top + b1_ref[...]
        a_scr[:, pl.ds(s * _PREP_N, _PREP_N)] = (
            jnp.dot(top, w2j, preferred_element_type=jnp.float32) + b2_ref[...])

    @pl.when((s >= _PREP_TILES) & (s < _PREP_TILES + _M_TILES))
    def _main():
        i = s - _PREP_TILES
        succ16 = succ_ref[...].astype(jnp.bfloat16)
        h = jnp.dot(succ16, w12b_scr[...], preferred_element_type=jnp.float32)
        pos = i * _TILE_M + jax.lax.broadcasted_iota(jnp.int32, (_TILE_M, 1), 0)
        onehot = ((pos >= culow_v_ref[...]) & (pos < cuhigh_v_ref[...])).astype(jnp.float32)
        h = h + jnp.dot(onehot, a_scr[...], preferred_element_type=jnp.float32)
        h = jnp.maximum(h, 0.0)
        l_col = jnp.sum(h * w3_ref[...], axis=1, keepdims=True)
        logits_ref[...] = l_col
        posT = i * _TILE_M + jax.lax.broadcasted_iota(jnp.int32, (1, _TILE_M), 1)
        onehotT = ((posT >= culow_c_ref[...]) & (posT < cuhigh_c_ref[...])).astype(jnp.float32)
        part = jnp.dot(onehotT, jnp.exp(l_col), preferred_element_type=jnp.float32)

        @pl.when(s == _PREP_TILES)
        def _init():
            psum_scr[...] = part

        @pl.when(s > _PREP_TILES)
        def _acc():
            psum_scr[...] = psum_scr[...] + part

    @pl.when(s == _STEPS - 1)
    def _recip():
        rtab_ref[...] = (1.0 / psum_scr[...]) * jnp.ones((_B, _B), jnp.float32)


def _sc_normalize_body(logits_hbm, pos_hbm, lob_hbm, hib_hbm, rtab_hbm,
                       out_hbm, x_v, pos_v, lob_v, hib_v, rt_v, sem):
    wid = lax.axis_index("s") * _NC + lax.axis_index("c")
    base = wid * _CHUNK
    d1 = pltpu.async_copy(logits_hbm.at[pl.ds(base, _CHUNK)], x_v, sem)
    d2 = pltpu.async_copy(pos_hbm.at[pl.ds(base, _CHUNK)], pos_v, sem)
    d3 = pltpu.async_copy(lob_hbm, lob_v, sem)
    d4 = pltpu.async_copy(hib_hbm, hib_v, sem)
    d5 = pltpu.async_copy(rtab_hbm, rt_v, sem)
    d1.wait()
    d2.wait()
    d3.wait()
    d4.wait()
    d5.wait()
    lob = [lob_v[pl.ds(b * _LANES, _LANES)] for b in range(_B)]
    hib = [hib_v[pl.ds(b * _LANES, _LANES)] for b in range(_B)]
    rrow = [rt_v[pl.ds(b * _LANES, _LANES)] for b in range(_B)]
    zero = jnp.full((_LANES,), 0.0, jnp.float32)
    for v in range(_NV):
        sl = pl.ds(v * _LANES, _LANES)
        pos = pos_v[sl]
        rv = zero
        for b in range(_B):
            m = (pos >= lob[b]) & (pos < hib[b])
            rv = jnp.where(m, rrow[b], rv)
        x_v[sl] = jnp.exp(x_v[sl]) * rv
    pltpu.sync_copy(x_v, out_hbm.at[pl.ds(base, _CHUNK)])


_sc_normalize = functools.partial(
    pl.kernel,
    out_type=jax.ShapeDtypeStruct((_TOTAL,), jnp.float32),
    mesh=plsc.VectorSubcoreMesh(core_axis_name="c", subcore_axis_name="s"),
    scratch_types=[
        pltpu.VMEM((_CHUNK,), jnp.float32),
        pltpu.VMEM((_CHUNK,), jnp.int32),
        pltpu.VMEM((_B * _LANES,), jnp.int32),
        pltpu.VMEM((_B * _LANES,), jnp.int32),
        pltpu.VMEM((_B * _LANES,), jnp.float32),
        pltpu.SemaphoreType.DMA,
    ],
)(_sc_normalize_body)


def kernel(current_embedding, successor_flat, cu_seqlens,
           W1, b1, W2, b2, W3, b3, Vw1, Vb1, Vw2, Vb2):
    del b3
    cu = cu_seqlens.astype(jnp.int32)
    culow_v = cu[:-1].reshape(1, _B)
    cuhigh_v = cu[1:].reshape(1, _B)
    culow_c = cu[:-1].reshape(_B, 1)
    cuhigh_c = cu[1:].reshape(_B, 1)
    b1r = b1.reshape(1, _TWOH)
    b2r = b2.reshape(1, _TWOH)
    w3r = W3.reshape(1, _TWOH)
    vb1r = Vb1.reshape(1, _H)
    vw2r = Vw2.reshape(1, _H)
    vb2r = Vb2.reshape(1, 1)

    logits_col, state_value, rtab = pl.pallas_call(
        _fused_kernel,
        grid=(_STEPS,),
        in_specs=[
            pl.BlockSpec((1, _B), lambda s: (0, 0)),
            pl.BlockSpec((1, _B), lambda s: (0, 0)),
            pl.BlockSpec((_B, 1), lambda s: (0, 0)),
            pl.BlockSpec((_B, 1), lambda s: (0, 0)),
            pl.BlockSpec((_TWOH, _TWOH), lambda s: (0, 0)),
            pl.BlockSpec((_TWOH, _PREP_N),
                         lambda s: (0, jnp.minimum(s, _PREP_TILES - 1))),
            pl.BlockSpec((_B, _H), lambda s: (0, 0)),
            pl.BlockSpec((1, _TWOH), lambda s: (0, 0)),
            pl.BlockSpec((1, _PREP_N),
                         lambda s: (0, jnp.minimum(s, _PREP_TILES - 1))),
            pl.BlockSpec((_TILE_M, _H),
                         lambda s: (jnp.clip(s - _PREP_TILES, 0, _M_TILES - 1), 0)),
            pl.BlockSpec((1, _TWOH), lambda s: (0, 0)),
            pl.BlockSpec((_H, _H), lambda s: (0, 0)),
            pl.BlockSpec((1, _H), lambda s: (0, 0)),
            pl.BlockSpec((1, _H), lambda s: (0, 0)),
            pl.BlockSpec(memory_space=pltpu.SMEM),
        ],
        out_specs=[
            pl.BlockSpec((_TILE_M, 1),
                         lambda s: (jnp.clip(s - _PREP_TILES, 0, _M_TILES - 1), 0)),
            pl.BlockSpec((_B, 1), lambda s: (0, 0)),
            pl.BlockSpec((_B, _B), lambda s: (0, 0)),
        ],
        out_shape=[
            jax.ShapeDtypeStruct((_TOTAL, 1), jnp.float32),
            jax.ShapeDtypeStruct((_B, 1), jnp.float32),
            jax.ShapeDtypeStruct((_B, _B), jnp.float32),
        ],
        scratch_shapes=[
            pltpu.VMEM((_H, _TWOH), jnp.bfloat16),
            pltpu.VMEM((_B, _TWOH), jnp.float32),
            pltpu.VMEM((_B, 1), jnp.float32),
        ],
    )(culow_v, cuhigh_v, culow_c, cuhigh_c, W1, W2, current_embedding,
      b1r, b2r, successor_flat, w3r, Vw1, vb1r, vw2r, vb2r)

    pos_tab = jnp.arange(_TOTAL, dtype=jnp.int32)
    lob_tab = jnp.repeat(cu[:-1], _LANES)
    hib_tab = jnp.repeat(cu[1:], _LANES)
    probs = _sc_normalize(logits_col.reshape(_TOTAL), pos_tab,
                          lob_tab, hib_tab, rtab.reshape(_B * _LANES))

    return probs, state_value
